# baseline (speedup 1.0000x reference)
import functools

import jax
import jax.numpy as jnp
import numpy as np
from jax.experimental import pallas as pl
from jax.experimental.pallas import tpu as pltpu

N_NODES = 10000
N_EDGES = 160000
HIDDEN = 256
N_PAD = 10240


def _pack_bf(x):
    h = x.shape[1] // 2
    b = lax.bitcast_convert_type(x, jnp.int32)
    lo16 = ((b[:, :h] + 0x8000) >> 16) & 0xFFFF
    hi16 = (b[:, h:] + 0x8000) & (-65536)
    return lo16 | hi16


def _unpack_bf(p):
    lo = lax.bitcast_convert_type(p << 16, jnp.float32)
    hi = lax.bitcast_convert_type(p & (-65536), jnp.float32)
    return lo, hi


def _lin1_body(x_ref, w_ref, b_ref, o_ref):
    o_ref[...] = jnp.dot(x_ref[...], w_ref[...],
                         preferred_element_type=jnp.float32) + b_ref[...]


def _lin1(x, w, b, rows_per_block=2048):
    n = x.shape[0]
    grid = n // rows_per_block
    return pl.pallas_call(
        _lin1_body,
        grid=(grid,),
        in_specs=[
            pl.BlockSpec((rows_per_block, x.shape[1]), lambda i: (i, 0)),
            pl.BlockSpec((x.shape[1], w.shape[1]), lambda i: (0, 0)),
            pl.BlockSpec((1, w.shape[1]), lambda i: (0, 0)),
        ],
        out_specs=pl.BlockSpec((rows_per_block, w.shape[1]), lambda i: (i, 0)),
        out_shape=jax.ShapeDtypeStruct((n, w.shape[1]), jnp.float32),
    )(x, w, b.reshape(1, -1))


def _deep_precomp(z, gamma, beta, wt, bt, wb, w2a, b2a, w2b, b2b,
                  rows_per_block=1024):
    n = z.shape[0]

    def body(z_ref, g_ref, be_ref, wt_ref, bt_ref, wb_ref, w2a_ref, b2a_ref,
             w2b_ref, b2b_ref, p_ref, q_ref, u_ref):
        z_ = z_ref[...]
        mu = jnp.mean(z_, axis=-1, keepdims=True)
        var = jnp.mean((z_ - mu) ** 2, axis=-1, keepdims=True)
        h = (z_ - mu) * jax.lax.rsqrt(var + 1e-5) * g_ref[...] + be_ref[...]
        h = jnp.maximum(h, 0.0)
        p_ref[...] = _pack_bf(jnp.dot(h, wt_ref[...],
                                      preferred_element_type=jnp.float32)
                              + bt_ref[...])
        q_ref[...] = _pack_bf(jnp.dot(h, wb_ref[...],
                                      preferred_element_type=jnp.float32))
        t = jnp.maximum(jnp.dot(h, w2a_ref[...],
                                preferred_element_type=jnp.float32)
                        + b2a_ref[...], 0.0)
        u_ref[...] = jnp.dot(t, w2b_ref[...],
                             preferred_element_type=jnp.float32) + b2b_ref[...]

    full = lambda a: pl.BlockSpec(a.shape, lambda i: tuple(0 for _ in a.shape))
    row = pl.BlockSpec((rows_per_block, HIDDEN), lambda i: (i, 0))
    halfrow = pl.BlockSpec((rows_per_block, HIDDEN // 2), lambda i: (i, 0))
    args = (z, gamma.reshape(1, -1), beta.reshape(1, -1), wt,
            bt.reshape(1, -1), wb, w2a, b2a.reshape(1, -1), w2b,
            b2b.reshape(1, -1))
    return pl.pallas_call(
        body,
        grid=(n // rows_per_block,),
        in_specs=[row] + [full(a) for a in args[1:]],
        out_specs=[halfrow, halfrow, row],
        out_shape=[jax.ShapeDtypeStruct((n, HIDDEN // 2), jnp.int32),
                   jax.ShapeDtypeStruct((n, HIDDEN // 2), jnp.int32),
                   jax.ShapeDtypeStruct((n, HIDDEN), jnp.float32)],
    )(*args)


def _edge_dense(ga, gb, ea_pad, we1, be1, we2, be2, wm2, bm2,
                edges_per_block=2000):
    n = ga.shape[0]

    def body(ga_ref, gb_ref, ea_ref, we1_ref, be1_ref, we2_ref, be2_ref,
             wm2_ref, bm2_ref, o_ref):
        t = jnp.maximum(jnp.dot(ea_ref[...], we1_ref[...],
                                preferred_element_type=jnp.float32)
                        + be1_ref[...], 0.0)
        gate = jnp.dot(t, we2_ref[...],
                       preferred_element_type=jnp.float32) + be2_ref[...]
        alo, ahi = _unpack_bf(ga_ref[...])
        blo, bhi = _unpack_bf(gb_ref[...])
        hlo = jnp.maximum(alo + blo, 0.0)
        hhi = jnp.maximum(ahi + bhi, 0.0)
        w2 = wm2_ref[...]
        m = (jnp.dot(hlo, w2[:HIDDEN // 2],
                     preferred_element_type=jnp.float32)
             + jnp.dot(hhi, w2[HIDDEN // 2:],
                       preferred_element_type=jnp.float32) + bm2_ref[...])
        o_ref[...] = gate * m

    full = lambda a: pl.BlockSpec(a.shape, lambda i: tuple(0 for _ in a.shape))
    row = pl.BlockSpec((edges_per_block, HIDDEN), lambda i: (i, 0))
    halfrow = pl.BlockSpec((edges_per_block, HIDDEN // 2), lambda i: (i, 0))
    args = (ea_pad, we1, be1.reshape(1, -1), we2, be2.reshape(1, -1),
            wm2, bm2.reshape(1, -1))
    return pl.pallas_call(
        body,
        grid=(n // edges_per_block,),
        in_specs=[halfrow, halfrow,
                  pl.BlockSpec((edges_per_block, ea_pad.shape[1]),
                               lambda i: (i, 0))] + [full(a) for a in args[1:]],
        out_specs=row,
        out_shape=jax.ShapeDtypeStruct((n, HIDDEN), jnp.float32),
    )(ga, gb, *args)


def _gated_edge(gk, gqv, edges_per_block=2000):
    n = gk.shape[0]
    H2 = HIDDEN // 2
    row = pl.BlockSpec((edges_per_block, HIDDEN), lambda i: (i, 0))

    def body(k_ref, qv_ref, o_ref):
        klo, khi = _unpack_bf(k_ref[...])
        qlo, qhi = _unpack_bf(qv_ref[:, :H2])
        vlo, vhi = _unpack_bf(qv_ref[:, H2:])
        o_ref[:, :H2] = jax.nn.sigmoid(klo + qlo) * vlo
        o_ref[:, H2:] = jax.nn.sigmoid(khi + qhi) * vhi

    return pl.pallas_call(
        body, grid=(n // edges_per_block,),
        in_specs=[pl.BlockSpec((edges_per_block, H2), lambda i: (i, 0)),
                  row],
        out_specs=row,
        out_shape=jax.ShapeDtypeStruct((n, HIDDEN), jnp.float32),
    )(gk, gqv)


def _gated_node(z, wk, bk, wq, bq, wv, bv, ws, rows_per_block=1024):
    n = z.shape[0]

    def body(z_ref, wk_ref, bk_ref, wq_ref, bq_ref, wv_ref, bv_ref, ws_ref,
             k_ref, qv_ref, s_ref):
        z_ = z_ref[...]
        k_ref[...] = _pack_bf(jnp.dot(z_, wk_ref[...],
                                      preferred_element_type=jnp.float32)
                              + bk_ref[...])
        qv_ref[:, :HIDDEN // 2] = _pack_bf(
            jnp.dot(z_, wq_ref[...], preferred_element_type=jnp.float32)
            + bq_ref[...])
        qv_ref[:, HIDDEN // 2:] = _pack_bf(
            jnp.dot(z_, wv_ref[...], preferred_element_type=jnp.float32)
            + bv_ref[...])
        s_ref[...] = jnp.dot(z_, ws_ref[...], preferred_element_type=jnp.float32)

    full = lambda a: pl.BlockSpec(a.shape, lambda i: tuple(0 for _ in a.shape))
    row = pl.BlockSpec((rows_per_block, HIDDEN), lambda i: (i, 0))
    halfrow = pl.BlockSpec((rows_per_block, HIDDEN // 2), lambda i: (i, 0))
    args = (wk, bk.reshape(1, -1), wq, bq.reshape(1, -1), wv,
            bv.reshape(1, -1), ws)
    return pl.pallas_call(
        body,
        grid=(n // rows_per_block,),
        in_specs=[row] + [full(a) for a in args],
        out_specs=[halfrow, row, row],
        out_shape=[jax.ShapeDtypeStruct((n, HIDDEN // 2), jnp.int32),
                   jax.ShapeDtypeStruct((n, HIDDEN), jnp.int32),
                   jax.ShapeDtypeStruct((n, HIDDEN), jnp.float32)],
    )(z, *args)


def _combine3(a, b, c, rows_per_block=2048):
    n = a.shape[0]
    row = pl.BlockSpec((rows_per_block, HIDDEN), lambda i: (i, 0))

    def body(a_ref, b_ref, c_ref, o_ref):
        o_ref[...] = a_ref[...] + b_ref[...] + c_ref[...]

    return pl.pallas_call(
        body, grid=(n // rows_per_block,),
        in_specs=[row, row, row], out_specs=row,
        out_shape=jax.ShapeDtypeStruct((n, HIDDEN), jnp.float32),
    )(a, b, c)


def _gated_combine(s, cnt, sk, bias, rows_per_block=2048):
    n = s.shape[0]
    row = pl.BlockSpec((rows_per_block, HIDDEN), lambda i: (i, 0))

    def body(s_ref, c_ref, sk_ref, b_ref, o_ref):
        o_ref[...] = (s_ref[...] / jnp.maximum(c_ref[..., 0:1], 1.0)
                      + sk_ref[...] + b_ref[...])

    return pl.pallas_call(
        body, grid=(n // rows_per_block,),
        in_specs=[row, pl.BlockSpec((rows_per_block, 16), lambda i: (i, 0)),
                  row, pl.BlockSpec((1, HIDDEN), lambda i: (0, 0))],
        out_specs=row,
        out_shape=jax.ShapeDtypeStruct((n, HIDDEN), jnp.float32),
    )(s, cnt, sk, bias.reshape(1, -1))


def _final(z, agg, u, w, b, rows_per_block=2048):
    n = z.shape[0]
    row = pl.BlockSpec((rows_per_block, HIDDEN), lambda i: (i, 0))

    def body(z_ref, a_ref, u_ref, w_ref, b_ref, o_ref):
        t = z_ref[...] + a_ref[...] + u_ref[...]
        o_ref[...] = jnp.dot(t, w_ref[...],
                             preferred_element_type=jnp.float32) + b_ref[...]

    return pl.pallas_call(
        body, grid=(n // rows_per_block,),
        in_specs=[row, row, row,
                  pl.BlockSpec(w.shape, lambda i: (0, 0)),
                  pl.BlockSpec((1, w.shape[1]), lambda i: (0, 0))],
        out_specs=pl.BlockSpec((rows_per_block, w.shape[1]), lambda i: (i, 0)),
        out_shape=jax.ShapeDtypeStruct((n, w.shape[1]), jnp.float32),
    )(z, agg, u, w, b.reshape(1, -1))


from jax import lax
from jax.experimental.pallas import tpu_sc as plsc

NW = 32
NPT = 313
BCH = 640
LIST_CAP = 162720
TRASH = NPT << 18
GC = 32
KGC = 64
NEGF = float("-inf")


def _vsm():
    return plsc.VectorSubcoreMesh(core_axis_name="c", subcore_axis_name="s")


def _nlp():
    return pltpu.CompilerParams(needs_layout_passes=False)


def _wid():
    return lax.axis_index("s") * 2 + lax.axis_index("c")


def _sc_bin(col):
    n_chunks = N_EDGES // BCH
    fl = BCH + 32

    @functools.partial(
        pl.kernel,
        out_type=[jax.ShapeDtypeStruct((NW * LIST_CAP,), jnp.int32),
                  jax.ShapeDtypeStruct((NW * 16,), jnp.int32)],
        mesh=_vsm(),
        compiler_params=_nlp(),
        scratch_types=[pltpu.VMEM((BCH,), jnp.int32),
                       pltpu.VMEM((fl + 16,), jnp.int32),
                       pltpu.VMEM((16,), jnp.int32)],
    )
    def k(col_hbm, lists_hbm, counts_hbm, colbuf, listbuf, cntbuf):
        w = _wid()
        lo = w * NPT
        base = pl.multiple_of(w * LIST_CAP, 8)
        trash = jnp.full((16,), TRASH, jnp.int32)
        lanes = lax.iota(jnp.int32, 16)

        def fill(t, _):
            listbuf[pl.ds(t * 16, 16)] = trash
            return 0

        def chunk(ci, gp):
            pltpu.sync_copy(col_hbm.at[pl.ds(ci * BCH, BCH)], colbuf)
            lax.fori_loop(0, fl // 16, fill, 0)

            def vec(j, wp):
                cv = colbuf[pl.ds(j * 16, 16)]
                m = (cv >= lo) & (cv < lo + NPT)
                eid = lanes + (ci * BCH + j * 16)
                packed = ((cv - lo) << 18) | eid
                c = plsc.cumsum(m.astype(jnp.int32))
                dst = jnp.where(m, wp + c - 1, fl)
                plsc.store_scatter(listbuf, [dst], packed)
                return wp + plsc.all_reduce_population_count(m)[0]
            wp = lax.fori_loop(0, BCH // 16, vec, 0)
            pltpu.sync_copy(listbuf.at[pl.ds(0, fl)],
                            lists_hbm.at[pl.ds(base + pl.multiple_of(gp, 8),
                                               fl)])
            return gp + ((wp + 7) & ~7)

        gp = lax.fori_loop(0, n_chunks, chunk, 0)
        lax.fori_loop(0, fl // 16, fill, 0)
        pltpu.sync_copy(listbuf.at[pl.ds(0, fl)],
                        lists_hbm.at[pl.ds(base + pl.multiple_of(gp, 8), fl)])
        n_pad = ((gp + GC - 1) // GC) * GC
        cntbuf[...] = jnp.full((16,), n_pad, jnp.int32)
        pltpu.sync_copy(cntbuf, counts_hbm.at[pl.ds(pl.multiple_of(w * 16, 8),
                                                    16)])

    return k(col)


def _sc_gather(tables, indices):
    nk = len(tables)
    n_chunks = N_EDGES // KGC
    trips = (((n_chunks + NW - 1) // NW) + 1) & ~1
    dims = [t.shape[1] for t in tables]
    dts = [t.dtype for t in tables]

    S = 2

    @functools.partial(
        pl.kernel,
        out_type=[jax.ShapeDtypeStruct((N_EDGES, dims[t]), dts[t])
                  for t in range(nk)],
        mesh=_vsm(),
        compiler_params=_nlp(),
        scratch_types=([pltpu.VMEM((KGC,), jnp.int32)] * (S * nk)
                       + [pltpu.VMEM((KGC, dims[t % nk]), dts[t % nk])
                          for t in range(S * nk)]
                       + [pltpu.SemaphoreType.DMA] * (S * nk)
                       + [pltpu.SemaphoreType.DMA] * (S * nk)),
    )
    def k(*refs):
        tabs = refs[:nk]
        idxs = refs[nk:2 * nk]
        outs = refs[2 * nk:3 * nk]
        rest = refs[3 * nk:]
        ibufs = [rest[s * nk:(s + 1) * nk] for s in range(S)]
        rest = rest[S * nk:]
        rbufs = [rest[s * nk:(s + 1) * nk] for s in range(S)]
        rest = rest[S * nk:]
        gsems = [rest[s * nk:(s + 1) * nk] for s in range(S)]
        rest = rest[S * nk:]
        wsems = [rest[s * nk:(s + 1) * nk] for s in range(S)]
        w = _wid()
        supers = (n_chunks + S * NW - 1) // (S * NW)

        def it(i, _):
            def coff(s):
                return (i * S + s) * NW + w

            def alive(s):
                return coff(s) < n_chunks

            for s in range(S):
                @pl.when(alive(s))
                def _(s=s):
                    off = pl.multiple_of(coff(s) * KGC, 8)
                    for t in range(nk):
                        pltpu.sync_copy(idxs[t].at[pl.ds(off, KGC)],
                                        ibufs[s][t])
            for s in range(S):
                @pl.when(alive(s))
                def _(s=s):
                    for t in range(nk):
                        pltpu.async_copy(tabs[t].at[ibufs[s][t]],
                                         rbufs[s][t], gsems[s][t])
            for s in range(S):
                @pl.when(alive(s))
                def _(s=s):
                    for t in range(nk):
                        pltpu.make_async_copy(tabs[t].at[ibufs[s][t]],
                                              rbufs[s][t],
                                              gsems[s][t]).wait()
                    off = pl.multiple_of(coff(s) * KGC, 8)
                    for t in range(nk):
                        pltpu.async_copy(rbufs[s][t],
                                         outs[t].at[pl.ds(off, KGC)],
                                         wsems[s][t])
            for s in range(S):
                @pl.when(alive(s))
                def _(s=s):
                    off = pl.multiple_of(coff(s) * KGC, 8)
                    for t in range(nk):
                        pltpu.make_async_copy(rbufs[s][t],
                                              outs[t].at[pl.ds(off, KGC)],
                                              wsems[s][t]).wait()
            return 0

        lax.fori_loop(0, supers, it, 0)

    return k(*tables, *indices)


def _sc_aggregate(msg, lists, counts, is_max):
    accw = (NPT + 1) * HIDDEN

    out_type = [jax.ShapeDtypeStruct((N_PAD * HIDDEN,), jnp.float32)]
    scratch = [pltpu.VMEM((accw,), jnp.float32),
               pltpu.VMEM((GC + 16,), jnp.int32),
               pltpu.VMEM((GC,), jnp.int32),
               pltpu.VMEM((GC, HIDDEN), jnp.float32),
               pltpu.VMEM((16,), jnp.int32),
               pltpu.SemaphoreType.DMA]
    if not is_max:
        out_type.append(jax.ShapeDtypeStruct((N_PAD * 16,), jnp.float32))
        scratch.insert(1, pltpu.VMEM(((NPT + 1) * 16,), jnp.float32))

    @functools.partial(pl.kernel, out_type=out_type, mesh=_vsm(),
                       compiler_params=_nlp(), scratch_types=scratch)
    def k(*refs):
        if is_max:
            (msg_hbm, lists_hbm, counts_hbm, agg_hbm,
             acc, pkbuf, idxbuf, rows, cbuf, sem) = refs
        else:
            (msg_hbm, lists_hbm, counts_hbm, agg_hbm, cnt_hbm,
             acc, cacc, pkbuf, idxbuf, rows, cbuf, sem) = refs
        w = _wid()
        lbase = pl.multiple_of(w * LIST_CAP, 8)
        init = NEGF if is_max else 0.0

        def ini(i, _):
            acc[pl.ds(i * 16, 16)] = jnp.full((16,), init, jnp.float32)
            return 0
        lax.fori_loop(0, accw // 16, ini, 0)
        if not is_max:
            def inic(i, _):
                cacc[pl.ds(i * 16, 16)] = jnp.zeros((16,), jnp.float32)
                return 0
            lax.fori_loop(0, (NPT + 1), inic, 0)

        pltpu.sync_copy(counts_hbm.at[pl.ds(pl.multiple_of(w * 16, 8), 16)],
                        cbuf)
        n_pad = cbuf[...][0]
        ones = jnp.ones((16,), jnp.float32)

        def chunk(j, _):
            pltpu.sync_copy(
                lists_hbm.at[pl.ds(lbase + pl.multiple_of(j * GC, 8), GC)],
                pkbuf.at[pl.ds(0, GC)])
            for t in range(GC // 16):
                idxbuf[pl.ds(t * 16, 16)] = pkbuf[pl.ds(t * 16, 16)] & 0x3FFFF
            pltpu.async_copy(msg_hbm.at[idxbuf], rows, sem).wait()

            def edge(i, _):
                pk = pkbuf[pl.ds(i, 16)][0]
                base = (pk >> 18) * HIDDEN
                if is_max:
                    for f in range(HIDDEN // 16):
                        sl = pl.ds(base + f * 16, 16)
                        acc[sl] = jnp.maximum(acc[sl],
                                              rows[i, pl.ds(f * 16, 16)])
                else:
                    for f in range(HIDDEN // 16):
                        plsc.addupdate(acc.at[pl.ds(base + f * 16, 16)],
                                       rows[i, pl.ds(f * 16, 16)])
                    plsc.addupdate(cacc.at[pl.ds((pk >> 18) * 16, 16)], ones)
                return 0
            lax.fori_loop(0, GC, edge, 0)
            return 0

        lax.fori_loop(0, n_pad // GC, chunk, 0)

        if is_max:
            def fin(i, _):
                v = acc[pl.ds(i * 16, 16)]
                acc[pl.ds(i * 16, 16)] = jnp.where(v == NEGF, 0.0, v)
                return 0
            lax.fori_loop(0, (NPT * HIDDEN) // 16, fin, 0)
        pltpu.sync_copy(
            acc.at[pl.ds(0, NPT * HIDDEN)],
            agg_hbm.at[pl.ds(pl.multiple_of(w * NPT * HIDDEN, 8),
                             NPT * HIDDEN)])
        if not is_max:
            pltpu.sync_copy(
                cacc.at[pl.ds(0, NPT * 16)],
                cnt_hbm.at[pl.ds(pl.multiple_of(w * NPT * 16, 8), NPT * 16)])

    if is_max:
        (agg,) = k(msg, lists, counts)
        return agg.reshape(N_PAD, HIDDEN)
    agg, cnt = k(msg, lists, counts)
    return agg.reshape(N_PAD, HIDDEN), cnt.reshape(N_PAD, 16)


def _tobf(a):
    return lax.bitcast_convert_type(
        a.astype(jnp.bfloat16).reshape(a.shape[0], -1, 2), jnp.int32)


def _frombf(a):
    return lax.bitcast_convert_type(a, jnp.bfloat16).reshape(a.shape[0], -1)


def kernel(x, edge_index, edge_attr, params):
    EDGE_IN = edge_attr.shape[1]
    row, col = edge_index[0], edge_index[1]
    xp = jnp.pad(x, ((0, N_PAD - N_NODES), (0, 0)))
    ea_pad = jnp.pad(edge_attr, ((0, 0), (0, 128 - edge_attr.shape[1])))

    lists, counts = _sc_bin(col)

    p_lin1 = params["lin1"]
    z = _lin1(xp, p_lin1["W"], p_lin1["b"])

    def deep_layer(z, lp):
        w1 = lp["node_mlp1"][0]["W"]
        p_, q_, u_ = _deep_precomp(
            z, lp["norm"]["gamma"], lp["norm"]["beta"],
            w1[:HIDDEN], lp["node_mlp1"][0]["b"], w1[HIDDEN:],
            lp["node_mlp2"][0]["W"], lp["node_mlp2"][0]["b"],
            lp["node_mlp2"][1]["W"], lp["node_mlp2"][1]["b"])
        ga, gb = _sc_gather([p_, q_], [row, col])
        we1 = jnp.pad(lp["edge_mlp"][0]["W"], ((0, 128 - EDGE_IN), (0, 0)))
        msg = _edge_dense(ga, gb, ea_pad, we1, lp["edge_mlp"][0]["b"],
                          lp["edge_mlp"][1]["W"], lp["edge_mlp"][1]["b"],
                          lp["node_mlp1"][1]["W"], lp["node_mlp1"][1]["b"])
        agg = _sc_aggregate(msg, lists, counts, is_max=True)
        return z, agg, u_

    z, agg, u = deep_layer(z, params["layers"][0])
    z = _combine3(z, agg, u)

    ap = params["att"][0]
    kp, qvp, sk = _gated_node(z, ap["key"]["W"], ap["key"]["b"],
                              ap["query"]["W"], ap["query"]["b"],
                              ap["value"]["W"], ap["value"]["b"],
                              ap["skip"]["W"])
    gk, gqv = _sc_gather([kp, qvp], [col, row])
    m = _gated_edge(gk, gqv)
    s, cnt = _sc_aggregate(m, lists, counts, is_max=False)
    z = _gated_combine(s, cnt, sk, ap["bias"])

    z, agg, u = deep_layer(z, params["layers"][1])
    out = _final(z, agg, u, params["lin2"]["W"], params["lin2"]["b"])
    return out[:N_NODES]


# speedup vs baseline: 1.0362x; 1.0362x over previous
import functools

import jax
import jax.numpy as jnp
import numpy as np
from jax.experimental import pallas as pl
from jax.experimental.pallas import tpu as pltpu

N_NODES = 10000
N_EDGES = 160000
HIDDEN = 256
N_PAD = 10240


def _pack_bf(x):
    h = x.shape[1] // 2
    b = lax.bitcast_convert_type(x, jnp.int32)
    lo16 = ((b[:, :h] + 0x8000) >> 16) & 0xFFFF
    hi16 = (b[:, h:] + 0x8000) & (-65536)
    return lo16 | hi16


def _unpack_bf(p):
    lo = lax.bitcast_convert_type(p << 16, jnp.float32)
    hi = lax.bitcast_convert_type(p & (-65536), jnp.float32)
    return lo, hi


def _lin1_body(x_ref, w_ref, b_ref, o_ref):
    o_ref[...] = jnp.dot(x_ref[...], w_ref[...],
                         preferred_element_type=jnp.float32) + b_ref[...]


def _lin1(x, w, b, rows_per_block=2048):
    n = x.shape[0]
    grid = n // rows_per_block
    return pl.pallas_call(
        _lin1_body,
        grid=(grid,),
        in_specs=[
            pl.BlockSpec((rows_per_block, x.shape[1]), lambda i: (i, 0)),
            pl.BlockSpec((x.shape[1], w.shape[1]), lambda i: (0, 0)),
            pl.BlockSpec((1, w.shape[1]), lambda i: (0, 0)),
        ],
        out_specs=pl.BlockSpec((rows_per_block, w.shape[1]), lambda i: (i, 0)),
        out_shape=jax.ShapeDtypeStruct((n, w.shape[1]), jnp.float32),
    )(x, w, b.reshape(1, -1))


def _deep_precomp(z, gamma, beta, wt, bt, wb, w2a, b2a, w2b, b2b,
                  rows_per_block=1024):
    n = z.shape[0]

    def body(z_ref, g_ref, be_ref, wt_ref, bt_ref, wb_ref, w2a_ref, b2a_ref,
             w2b_ref, b2b_ref, p_ref, q_ref, u_ref):
        z_ = z_ref[...]
        mu = jnp.mean(z_, axis=-1, keepdims=True)
        var = jnp.mean((z_ - mu) ** 2, axis=-1, keepdims=True)
        h = (z_ - mu) * jax.lax.rsqrt(var + 1e-5) * g_ref[...] + be_ref[...]
        h = jnp.maximum(h, 0.0)
        p_ref[...] = _pack_bf(jnp.dot(h, wt_ref[...],
                                      preferred_element_type=jnp.float32)
                              + bt_ref[...])
        q_ref[...] = _pack_bf(jnp.dot(h, wb_ref[...],
                                      preferred_element_type=jnp.float32))
        t = jnp.maximum(jnp.dot(h, w2a_ref[...],
                                preferred_element_type=jnp.float32)
                        + b2a_ref[...], 0.0)
        u_ref[...] = jnp.dot(t, w2b_ref[...],
                             preferred_element_type=jnp.float32) + b2b_ref[...]

    full = lambda a: pl.BlockSpec(a.shape, lambda i: tuple(0 for _ in a.shape))
    row = pl.BlockSpec((rows_per_block, HIDDEN), lambda i: (i, 0))
    halfrow = pl.BlockSpec((rows_per_block, HIDDEN // 2), lambda i: (i, 0))
    args = (z, gamma.reshape(1, -1), beta.reshape(1, -1), wt,
            bt.reshape(1, -1), wb, w2a, b2a.reshape(1, -1), w2b,
            b2b.reshape(1, -1))
    return pl.pallas_call(
        body,
        grid=(n // rows_per_block,),
        in_specs=[row] + [full(a) for a in args[1:]],
        out_specs=[halfrow, halfrow, row],
        out_shape=[jax.ShapeDtypeStruct((n, HIDDEN // 2), jnp.int32),
                   jax.ShapeDtypeStruct((n, HIDDEN // 2), jnp.int32),
                   jax.ShapeDtypeStruct((n, HIDDEN), jnp.float32)],
    )(*args)


def _edge_dense(ga, gb, ea_pad, we1, be1, we2, be2, wm2, bm2,
                edges_per_block=2000):
    n = ga.shape[0]

    def body(ga_ref, gb_ref, ea_ref, we1_ref, be1_ref, we2_ref, be2_ref,
             wm2_ref, bm2_ref, o_ref):
        t = jnp.maximum(jnp.dot(ea_ref[...], we1_ref[...],
                                preferred_element_type=jnp.float32)
                        + be1_ref[...], 0.0)
        gate = jnp.dot(t, we2_ref[...],
                       preferred_element_type=jnp.float32) + be2_ref[...]
        alo, ahi = _unpack_bf(ga_ref[...])
        blo, bhi = _unpack_bf(gb_ref[...])
        hlo = jnp.maximum(alo + blo, 0.0)
        hhi = jnp.maximum(ahi + bhi, 0.0)
        w2 = wm2_ref[...]
        m = (jnp.dot(hlo, w2[:HIDDEN // 2],
                     preferred_element_type=jnp.float32)
             + jnp.dot(hhi, w2[HIDDEN // 2:],
                       preferred_element_type=jnp.float32) + bm2_ref[...])
        o_ref[...] = gate * m

    full = lambda a: pl.BlockSpec(a.shape, lambda i: tuple(0 for _ in a.shape))
    row = pl.BlockSpec((edges_per_block, HIDDEN), lambda i: (i, 0))
    halfrow = pl.BlockSpec((edges_per_block, HIDDEN // 2), lambda i: (i, 0))
    args = (ea_pad, we1, be1.reshape(1, -1), we2, be2.reshape(1, -1),
            wm2, bm2.reshape(1, -1))
    return pl.pallas_call(
        body,
        grid=(n // edges_per_block,),
        in_specs=[halfrow, halfrow,
                  pl.BlockSpec((edges_per_block, ea_pad.shape[1]),
                               lambda i: (i, 0))] + [full(a) for a in args[1:]],
        out_specs=row,
        out_shape=jax.ShapeDtypeStruct((n, HIDDEN), jnp.float32),
    )(ga, gb, *args)


def _gated_edge(gk, gqv, edges_per_block=2000):
    n = gk.shape[0]
    H2 = HIDDEN // 2
    row = pl.BlockSpec((edges_per_block, HIDDEN), lambda i: (i, 0))

    def body(k_ref, qv_ref, o_ref):
        klo, khi = _unpack_bf(k_ref[...])
        qlo, qhi = _unpack_bf(qv_ref[:, :H2])
        vlo, vhi = _unpack_bf(qv_ref[:, H2:])
        o_ref[:, :H2] = jax.nn.sigmoid(klo + qlo) * vlo
        o_ref[:, H2:] = jax.nn.sigmoid(khi + qhi) * vhi

    return pl.pallas_call(
        body, grid=(n // edges_per_block,),
        in_specs=[pl.BlockSpec((edges_per_block, H2), lambda i: (i, 0)),
                  row],
        out_specs=row,
        out_shape=jax.ShapeDtypeStruct((n, HIDDEN), jnp.float32),
    )(gk, gqv)


def _gated_node(z, wk, bk, wq, bq, wv, bv, ws, rows_per_block=1024):
    n = z.shape[0]

    def body(z_ref, wk_ref, bk_ref, wq_ref, bq_ref, wv_ref, bv_ref, ws_ref,
             k_ref, qv_ref, s_ref):
        z_ = z_ref[...]
        k_ref[...] = _pack_bf(jnp.dot(z_, wk_ref[...],
                                      preferred_element_type=jnp.float32)
                              + bk_ref[...])
        qv_ref[:, :HIDDEN // 2] = _pack_bf(
            jnp.dot(z_, wq_ref[...], preferred_element_type=jnp.float32)
            + bq_ref[...])
        qv_ref[:, HIDDEN // 2:] = _pack_bf(
            jnp.dot(z_, wv_ref[...], preferred_element_type=jnp.float32)
            + bv_ref[...])
        s_ref[...] = jnp.dot(z_, ws_ref[...], preferred_element_type=jnp.float32)

    full = lambda a: pl.BlockSpec(a.shape, lambda i: tuple(0 for _ in a.shape))
    row = pl.BlockSpec((rows_per_block, HIDDEN), lambda i: (i, 0))
    halfrow = pl.BlockSpec((rows_per_block, HIDDEN // 2), lambda i: (i, 0))
    args = (wk, bk.reshape(1, -1), wq, bq.reshape(1, -1), wv,
            bv.reshape(1, -1), ws)
    return pl.pallas_call(
        body,
        grid=(n // rows_per_block,),
        in_specs=[row] + [full(a) for a in args],
        out_specs=[halfrow, row, row],
        out_shape=[jax.ShapeDtypeStruct((n, HIDDEN // 2), jnp.int32),
                   jax.ShapeDtypeStruct((n, HIDDEN), jnp.int32),
                   jax.ShapeDtypeStruct((n, HIDDEN), jnp.float32)],
    )(z, *args)


def _combine3(a, b, c, rows_per_block=2048):
    n = a.shape[0]
    row = pl.BlockSpec((rows_per_block, HIDDEN), lambda i: (i, 0))

    def body(a_ref, b_ref, c_ref, o_ref):
        o_ref[...] = a_ref[...] + b_ref[...] + c_ref[...]

    return pl.pallas_call(
        body, grid=(n // rows_per_block,),
        in_specs=[row, row, row], out_specs=row,
        out_shape=jax.ShapeDtypeStruct((n, HIDDEN), jnp.float32),
    )(a, b, c)


def _gated_combine(s, cnt, sk, bias, rows_per_block=2048):
    n = s.shape[0]
    row = pl.BlockSpec((rows_per_block, HIDDEN), lambda i: (i, 0))

    def body(s_ref, c_ref, sk_ref, b_ref, o_ref):
        o_ref[...] = (s_ref[...] / jnp.maximum(c_ref[..., 0:1], 1.0)
                      + sk_ref[...] + b_ref[...])

    return pl.pallas_call(
        body, grid=(n // rows_per_block,),
        in_specs=[row, pl.BlockSpec((rows_per_block, 16), lambda i: (i, 0)),
                  row, pl.BlockSpec((1, HIDDEN), lambda i: (0, 0))],
        out_specs=row,
        out_shape=jax.ShapeDtypeStruct((n, HIDDEN), jnp.float32),
    )(s, cnt, sk, bias.reshape(1, -1))


def _final(z, agg, u, w, b, rows_per_block=2048):
    n = z.shape[0]
    row = pl.BlockSpec((rows_per_block, HIDDEN), lambda i: (i, 0))

    def body(z_ref, a_ref, u_ref, w_ref, b_ref, o_ref):
        t = z_ref[...] + a_ref[...] + u_ref[...]
        o_ref[...] = jnp.dot(t, w_ref[...],
                             preferred_element_type=jnp.float32) + b_ref[...]

    return pl.pallas_call(
        body, grid=(n // rows_per_block,),
        in_specs=[row, row, row,
                  pl.BlockSpec(w.shape, lambda i: (0, 0)),
                  pl.BlockSpec((1, w.shape[1]), lambda i: (0, 0))],
        out_specs=pl.BlockSpec((rows_per_block, w.shape[1]), lambda i: (i, 0)),
        out_shape=jax.ShapeDtypeStruct((n, w.shape[1]), jnp.float32),
    )(z, agg, u, w, b.reshape(1, -1))


from jax import lax
from jax.experimental.pallas import tpu_sc as plsc

NW = 32
NPT = 313
BCH = 640
LIST_CAP = 162720
TRASH = NPT << 18
GC = 32
KGC = 64
NEGF = float("-inf")


def _vsm():
    return plsc.VectorSubcoreMesh(core_axis_name="c", subcore_axis_name="s")


def _nlp():
    return pltpu.CompilerParams(needs_layout_passes=False)


def _wid():
    return lax.axis_index("s") * 2 + lax.axis_index("c")


def _sc_bin(col):
    n_chunks = N_EDGES // BCH
    fl = BCH + 32

    @functools.partial(
        pl.kernel,
        out_type=[jax.ShapeDtypeStruct((NW * LIST_CAP,), jnp.int32),
                  jax.ShapeDtypeStruct((NW * 16,), jnp.int32)],
        mesh=_vsm(),
        compiler_params=_nlp(),
        scratch_types=[pltpu.VMEM((BCH,), jnp.int32),
                       pltpu.VMEM((fl + 16,), jnp.int32),
                       pltpu.VMEM((16,), jnp.int32)],
    )
    def k(col_hbm, lists_hbm, counts_hbm, colbuf, listbuf, cntbuf):
        w = _wid()
        lo = w * NPT
        base = pl.multiple_of(w * LIST_CAP, 8)
        trash = jnp.full((16,), TRASH, jnp.int32)
        lanes = lax.iota(jnp.int32, 16)

        def fill(t, _):
            listbuf[pl.ds(t * 16, 16)] = trash
            return 0

        def chunk(ci, gp):
            pltpu.sync_copy(col_hbm.at[pl.ds(ci * BCH, BCH)], colbuf)
            lax.fori_loop(0, fl // 16, fill, 0)

            def vec(j, wp):
                cv = colbuf[pl.ds(j * 16, 16)]
                m = (cv >= lo) & (cv < lo + NPT)
                eid = lanes + (ci * BCH + j * 16)
                packed = ((cv - lo) << 18) | eid
                c = plsc.cumsum(m.astype(jnp.int32))
                dst = jnp.where(m, wp + c - 1, fl)
                plsc.store_scatter(listbuf, [dst], packed)
                return wp + plsc.all_reduce_population_count(m)[0]
            wp = lax.fori_loop(0, BCH // 16, vec, 0)
            pltpu.sync_copy(listbuf.at[pl.ds(0, fl)],
                            lists_hbm.at[pl.ds(base + pl.multiple_of(gp, 8),
                                               fl)])
            return gp + ((wp + 7) & ~7)

        gp = lax.fori_loop(0, n_chunks, chunk, 0)
        lax.fori_loop(0, fl // 16, fill, 0)
        pltpu.sync_copy(listbuf.at[pl.ds(0, fl)],
                        lists_hbm.at[pl.ds(base + pl.multiple_of(gp, 8), fl)])
        n_pad = ((gp + GC - 1) // GC) * GC
        cntbuf[...] = jnp.full((16,), n_pad, jnp.int32)
        pltpu.sync_copy(cntbuf, counts_hbm.at[pl.ds(pl.multiple_of(w * 16, 8),
                                                    16)])

    return k(col)


def _sc_gather(tables, indices):
    nk = len(tables)
    n_chunks = N_EDGES // KGC
    trips = (((n_chunks + NW - 1) // NW) + 1) & ~1
    dims = [t.shape[1] for t in tables]
    dts = [t.dtype for t in tables]

    @functools.partial(
        pl.kernel,
        out_type=[jax.ShapeDtypeStruct((N_EDGES, dims[t]), dts[t])
                  for t in range(nk)],
        mesh=_vsm(),
        compiler_params=_nlp(),
        scratch_types=([pltpu.VMEM((KGC,), jnp.int32)] * (2 * nk)
                       + [pltpu.VMEM((KGC, dims[t % nk]), dts[t % nk])
                          for t in range(2 * nk)]
                       + [pltpu.SemaphoreType.DMA] * (2 * nk)
                       + [pltpu.SemaphoreType.DMA] * (2 * nk)),
    )
    def k(*refs):
        tabs = refs[:nk]
        idxs = refs[nk:2 * nk]
        outs = refs[2 * nk:3 * nk]
        rest = refs[3 * nk:]
        ibufs = [rest[0:nk], rest[nk:2 * nk]]
        rbufs = [rest[2 * nk:3 * nk], rest[3 * nk:4 * nk]]
        gsems = [rest[4 * nk:5 * nk], rest[5 * nk:6 * nk]]
        wsems = [rest[6 * nk:7 * nk], rest[7 * nk:8 * nk]]
        w = _wid()

        def chunk_of(i):
            return i * NW + w

        def live(i):
            c = chunk_of(i)
            return (i >= 0) & (c < n_chunks)

        def load_idx_and_gather(i, p):
            @pl.when(live(i))
            def _():
                off = pl.multiple_of(chunk_of(i) * KGC, 8)
                for t in range(nk):
                    pltpu.sync_copy(idxs[t].at[pl.ds(off, KGC)], ibufs[p][t])
                for t in range(nk):
                    pltpu.async_copy(tabs[t].at[ibufs[p][t]], rbufs[p][t],
                                     gsems[p][t])

        def wait_gather(i, p):
            @pl.when(live(i))
            def _():
                for t in range(nk):
                    pltpu.make_async_copy(tabs[t].at[ibufs[p][t]],
                                          rbufs[p][t], gsems[p][t]).wait()

        def start_writeback(i, p):
            @pl.when(live(i))
            def _():
                off = pl.multiple_of(chunk_of(i) * KGC, 8)
                for t in range(nk):
                    pltpu.async_copy(rbufs[p][t], outs[t].at[pl.ds(off, KGC)],
                                     wsems[p][t])

        def wait_writeback(i, p):
            @pl.when(live(i))
            def _():
                off = pl.multiple_of(chunk_of(i) * KGC, 8)
                for t in range(nk):
                    pltpu.make_async_copy(rbufs[p][t],
                                          outs[t].at[pl.ds(off, KGC)],
                                          wsems[p][t]).wait()

        load_idx_and_gather(0, 0)

        def it(i2, _):
            for p in range(2):
                i = i2 * 2 + p
                wait_writeback(i - 1, 1 - p)
                load_idx_and_gather(i + 1, 1 - p)
                wait_gather(i, p)
                start_writeback(i, p)
            return 0

        lax.fori_loop(0, trips // 2, it, 0)
        wait_writeback(trips - 1, (trips - 1) % 2)

    return k(*tables, *indices)


def _sc_aggregate(msg, lists, counts, is_max):
    accw = (NPT + 1) * HIDDEN

    out_type = [jax.ShapeDtypeStruct((N_PAD * HIDDEN,), jnp.float32)]
    scratch = [pltpu.VMEM((accw,), jnp.float32),
               pltpu.VMEM((GC + 16,), jnp.int32),
               pltpu.VMEM((GC,), jnp.int32),
               pltpu.VMEM((GC, HIDDEN), jnp.float32),
               pltpu.VMEM((16,), jnp.int32),
               pltpu.SemaphoreType.DMA]
    if not is_max:
        out_type.append(jax.ShapeDtypeStruct((N_PAD * 16,), jnp.float32))
        scratch.insert(1, pltpu.VMEM(((NPT + 1) * 16,), jnp.float32))

    @functools.partial(pl.kernel, out_type=out_type, mesh=_vsm(),
                       compiler_params=_nlp(), scratch_types=scratch)
    def k(*refs):
        if is_max:
            (msg_hbm, lists_hbm, counts_hbm, agg_hbm,
             acc, pkbuf, idxbuf, rows, cbuf, sem) = refs
        else:
            (msg_hbm, lists_hbm, counts_hbm, agg_hbm, cnt_hbm,
             acc, cacc, pkbuf, idxbuf, rows, cbuf, sem) = refs
        w = _wid()
        lbase = pl.multiple_of(w * LIST_CAP, 8)
        init = NEGF if is_max else 0.0

        def ini(i, _):
            acc[pl.ds(i * 16, 16)] = jnp.full((16,), init, jnp.float32)
            return 0
        lax.fori_loop(0, accw // 16, ini, 0)
        if not is_max:
            def inic(i, _):
                cacc[pl.ds(i * 16, 16)] = jnp.zeros((16,), jnp.float32)
                return 0
            lax.fori_loop(0, (NPT + 1), inic, 0)

        pltpu.sync_copy(counts_hbm.at[pl.ds(pl.multiple_of(w * 16, 8), 16)],
                        cbuf)
        n_pad = cbuf[...][0]
        ones = jnp.ones((16,), jnp.float32)

        def chunk(j, _):
            pltpu.sync_copy(
                lists_hbm.at[pl.ds(lbase + pl.multiple_of(j * GC, 8), GC)],
                pkbuf.at[pl.ds(0, GC)])
            for t in range(GC // 16):
                idxbuf[pl.ds(t * 16, 16)] = pkbuf[pl.ds(t * 16, 16)] & 0x3FFFF
            pltpu.async_copy(msg_hbm.at[idxbuf], rows, sem).wait()

            def edge(i, _):
                pk = pkbuf[pl.ds(i, 16)][0]
                base = (pk >> 18) * HIDDEN
                if is_max:
                    for f in range(HIDDEN // 16):
                        sl = pl.ds(base + f * 16, 16)
                        acc[sl] = jnp.maximum(acc[sl],
                                              rows[i, pl.ds(f * 16, 16)])
                else:
                    for f in range(HIDDEN // 16):
                        plsc.addupdate(acc.at[pl.ds(base + f * 16, 16)],
                                       rows[i, pl.ds(f * 16, 16)])
                    plsc.addupdate(cacc.at[pl.ds((pk >> 18) * 16, 16)], ones)
                return 0
            lax.fori_loop(0, GC, edge, 0)
            return 0

        lax.fori_loop(0, n_pad // GC, chunk, 0)

        if is_max:
            def fin(i, _):
                v = acc[pl.ds(i * 16, 16)]
                acc[pl.ds(i * 16, 16)] = jnp.where(v == NEGF, 0.0, v)
                return 0
            lax.fori_loop(0, (NPT * HIDDEN) // 16, fin, 0)
        pltpu.sync_copy(
            acc.at[pl.ds(0, NPT * HIDDEN)],
            agg_hbm.at[pl.ds(pl.multiple_of(w * NPT * HIDDEN, 8),
                             NPT * HIDDEN)])
        if not is_max:
            pltpu.sync_copy(
                cacc.at[pl.ds(0, NPT * 16)],
                cnt_hbm.at[pl.ds(pl.multiple_of(w * NPT * 16, 8), NPT * 16)])

    if is_max:
        (agg,) = k(msg, lists, counts)
        return agg.reshape(N_PAD, HIDDEN)
    agg, cnt = k(msg, lists, counts)
    return agg.reshape(N_PAD, HIDDEN), cnt.reshape(N_PAD, 16)


def _tobf(a):
    return lax.bitcast_convert_type(
        a.astype(jnp.bfloat16).reshape(a.shape[0], -1, 2), jnp.int32)


def _frombf(a):
    return lax.bitcast_convert_type(a, jnp.bfloat16).reshape(a.shape[0], -1)


def kernel(x, edge_index, edge_attr, params):
    EDGE_IN = edge_attr.shape[1]
    row, col = edge_index[0], edge_index[1]
    xp = jnp.pad(x, ((0, N_PAD - N_NODES), (0, 0)))
    ea_pad = jnp.pad(edge_attr, ((0, 0), (0, 128 - edge_attr.shape[1])))

    lists, counts = _sc_bin(col)

    p_lin1 = params["lin1"]
    z = _lin1(xp, p_lin1["W"], p_lin1["b"])

    def deep_layer(z, lp):
        w1 = lp["node_mlp1"][0]["W"]
        p_, q_, u_ = _deep_precomp(
            z, lp["norm"]["gamma"], lp["norm"]["beta"],
            w1[:HIDDEN], lp["node_mlp1"][0]["b"], w1[HIDDEN:],
            lp["node_mlp2"][0]["W"], lp["node_mlp2"][0]["b"],
            lp["node_mlp2"][1]["W"], lp["node_mlp2"][1]["b"])
        ga, gb = _sc_gather([p_, q_], [row, col])
        we1 = jnp.pad(lp["edge_mlp"][0]["W"], ((0, 128 - EDGE_IN), (0, 0)))
        msg = _edge_dense(ga, gb, ea_pad, we1, lp["edge_mlp"][0]["b"],
                          lp["edge_mlp"][1]["W"], lp["edge_mlp"][1]["b"],
                          lp["node_mlp1"][1]["W"], lp["node_mlp1"][1]["b"])
        agg = _sc_aggregate(msg, lists, counts, is_max=True)
        return z, agg, u_

    z, agg, u = deep_layer(z, params["layers"][0])
    z = _combine3(z, agg, u)

    ap = params["att"][0]
    kp, qvp, sk = _gated_node(z, ap["key"]["W"], ap["key"]["b"],
                              ap["query"]["W"], ap["query"]["b"],
                              ap["value"]["W"], ap["value"]["b"],
                              ap["skip"]["W"])
    gk, gqv = _sc_gather([kp, qvp], [col, row])
    m = _gated_edge(gk, gqv)
    s, cnt = _sc_aggregate(m, lists, counts, is_max=False)
    z = _gated_combine(s, cnt, sk, ap["bias"])

    z, agg, u = deep_layer(z, params["layers"][1])
    out = _final(z, agg, u, params["lin2"]["W"], params["lin2"]["b"])
    return out[:N_NODES]
